# baseline (device time: 90443 ns/iter reference)
import functools

import jax
import jax.numpy as jnp
from jax import lax
from jax.experimental import pallas as pl
from jax.experimental.pallas import tpu as pltpu

N_RING = 8
M = 2048
N = 2048
K = 8192
TILE_M = M // N_RING
N_FWD = 4
N_BWD = 3
N_CHUNK = 4
NC = N // N_CHUNK
W_SUB = 256
SUB_PER_CHUNK = NC // W_SUB
N_SUB = N // W_SUB


def _ring_pos(x, z):
    return jnp.where(x == 0, z, (N_RING - 1) - z)


def _coords_of(rp):
    x = (rp >= 4).astype(rp.dtype)
    z = jnp.where(rp < 4, rp, (N_RING - 1) - rp)
    return x, z


def kernel(dy, W):
    my_x = lax.axis_index("x")
    my_z = lax.axis_index("z")
    pos = _ring_pos(my_x, my_z)

    dy_tile = lax.dynamic_slice_in_dim(dy, pos * TILE_M, TILE_M, axis=0)
    dy_bf = dy_tile.astype(jnp.bfloat16)

    def body(dy_ref, w_hbm, out_ref,
             w_buf, w_bf, partial,
             y_send, y_recv, red_bf, f_recv, b_recv,
             w_sems,
             y_send_sems, y_recv_sems,
             f_send_sems, f_recv_sems, b_send_sems, b_recv_sems):
        x = lax.axis_index("x")
        y = lax.axis_index("y")
        z = lax.axis_index("z")
        rp = _ring_pos(x, z)
        rx, rz = _coords_of((rp + 1) % N_RING)
        lx, lz = _coords_of((rp - 1) % N_RING)
        right = (rx, y, rz)
        left = (lx, y, lz)
        partner = (x, 1 - y, z)

        w_dmas = [None] * N_SUB
        w_dmas[0] = pltpu.make_async_copy(
            w_hbm.at[pl.ds(0, W_SUB)], w_buf.at[0], w_sems.at[0])
        w_dmas[0].start()

        barrier_sem = pltpu.get_barrier_semaphore()
        for nbr in (left, right, partner):
            pl.semaphore_signal(
                barrier_sem, inc=1,
                device_id=nbr, device_id_type=pl.DeviceIdType.MESH,
            )
        pl.semaphore_wait(barrier_sem, 3)

        def mk(src, dst, ssem, rsem, dev):
            return pltpu.make_async_remote_copy(
                src_ref=src, dst_ref=dst, send_sem=ssem, recv_sem=rsem,
                device_id=dev, device_id_type=pl.DeviceIdType.MESH,
            )

        y_rdmas = [None] * N_CHUNK

        for s in range(N_SUB):
            w_dmas[s].wait()
            if s + 1 < N_SUB:
                w_dmas[s + 1] = pltpu.make_async_copy(
                    w_hbm.at[pl.ds((s + 1) * W_SUB, W_SUB)],
                    w_buf.at[(s + 1) % 2], w_sems.at[(s + 1) % 2])
                w_dmas[s + 1].start()
            w_bf[...] = w_buf[s % 2].astype(jnp.bfloat16)
            partial[:, pl.ds(s * W_SUB, W_SUB)] = lax.dot_general(
                dy_ref[...], w_bf[...],
                dimension_numbers=(((1,), (1,)), ((), ())),
                preferred_element_type=jnp.float32,
            )
            if (s + 1) % SUB_PER_CHUNK == 0:
                c = (s + 1) // SUB_PER_CHUNK - 1
                y_send[c] = partial[:, pl.ds(c * NC, NC)].astype(jnp.bfloat16)
                r = mk(y_send.at[c], y_recv.at[c],
                       y_send_sems.at[c], y_recv_sems.at[c], partner)
                r.start()
                y_rdmas[c] = r

        f_rdmas = [[None] * N_CHUNK for _ in range(N_FWD)]
        b_rdmas = [[None] * N_CHUNK for _ in range(N_BWD)]

        for c in range(N_CHUNK):
            y_rdmas[c].wait_recv()
            red_f32 = (partial[:, pl.ds(c * NC, NC)]
                       + y_recv[c].astype(jnp.float32))
            out_ref[pl.ds(rp * TILE_M, TILE_M), pl.ds(c * NC, NC)] = red_f32
            red_bf[c] = red_f32.astype(jnp.bfloat16)
            fr = mk(red_bf.at[c], f_recv.at[0, c],
                    f_send_sems.at[0, c], f_recv_sems.at[0, c], right)
            fr.start()
            f_rdmas[0][c] = fr
            br = mk(red_bf.at[c], b_recv.at[0, c],
                    b_send_sems.at[0, c], b_recv_sems.at[0, c], left)
            br.start()
            b_rdmas[0][c] = br

        for h in range(N_FWD):
            f_origin = (rp - 1 - h) % N_RING
            b_origin = (rp + 1 + h) % N_RING
            for c in range(N_CHUNK):
                f_rdmas[h][c].wait_recv()
                if h + 1 < N_FWD:
                    nxt = mk(f_recv.at[h, c], f_recv.at[h + 1, c],
                             f_send_sems.at[h + 1, c],
                             f_recv_sems.at[h + 1, c], right)
                    nxt.start()
                    f_rdmas[h + 1][c] = nxt
                out_ref[pl.ds(f_origin * TILE_M, TILE_M),
                        pl.ds(c * NC, NC)] = f_recv[h, c].astype(jnp.float32)

                if h < N_BWD:
                    b_rdmas[h][c].wait_recv()
                    if h + 1 < N_BWD:
                        nxt = mk(b_recv.at[h, c], b_recv.at[h + 1, c],
                                 b_send_sems.at[h + 1, c],
                                 b_recv_sems.at[h + 1, c], left)
                        nxt.start()
                        b_rdmas[h + 1][c] = nxt
                    out_ref[pl.ds(b_origin * TILE_M, TILE_M),
                            pl.ds(c * NC, NC)] = (
                        b_recv[h, c].astype(jnp.float32))

        for r in y_rdmas:
            r.wait_send()
        for row in f_rdmas + b_rdmas:
            for r in row:
                r.wait_send()

        @functools.partial(pl.run_scoped,
                           exit_sem=pltpu.SemaphoreType.REGULAR)
        def _(exit_sem):
            for nbr in (left, right, partner):
                pl.semaphore_signal(
                    exit_sem, inc=1,
                    device_id=nbr, device_id_type=pl.DeviceIdType.MESH,
                )
            pl.semaphore_wait(exit_sem, 3)

    return pl.pallas_call(
        body,
        out_shape=jax.ShapeDtypeStruct((M, N), jnp.float32),
        in_specs=[
            pl.BlockSpec(memory_space=pltpu.VMEM),
            pl.BlockSpec(memory_space=pltpu.MemorySpace.HBM),
        ],
        out_specs=pl.BlockSpec(memory_space=pltpu.VMEM),
        scratch_shapes=[
            pltpu.VMEM((2, W_SUB, K), jnp.float32),
            pltpu.VMEM((W_SUB, K), jnp.bfloat16),
            pltpu.VMEM((TILE_M, N), jnp.float32),
            pltpu.VMEM((N_CHUNK, TILE_M, NC), jnp.bfloat16),
            pltpu.VMEM((N_CHUNK, TILE_M, NC), jnp.bfloat16),
            pltpu.VMEM((N_CHUNK, TILE_M, NC), jnp.bfloat16),
            pltpu.VMEM((N_FWD, N_CHUNK, TILE_M, NC), jnp.bfloat16),
            pltpu.VMEM((N_BWD, N_CHUNK, TILE_M, NC), jnp.bfloat16),
            pltpu.SemaphoreType.DMA((2,)),
            pltpu.SemaphoreType.DMA((N_CHUNK,)),
            pltpu.SemaphoreType.DMA((N_CHUNK,)),
            pltpu.SemaphoreType.DMA((N_FWD, N_CHUNK)),
            pltpu.SemaphoreType.DMA((N_FWD, N_CHUNK)),
            pltpu.SemaphoreType.DMA((N_BWD, N_CHUNK)),
            pltpu.SemaphoreType.DMA((N_BWD, N_CHUNK)),
        ],
        compiler_params=pltpu.CompilerParams(collective_id=0),
    )(dy_bf, W)


# device time: 86951 ns/iter; 1.0402x vs baseline; 1.0402x over previous
import functools

import jax
import jax.numpy as jnp
from jax import lax
from jax.experimental import pallas as pl
from jax.experimental.pallas import tpu as pltpu

N_RING = 8
M = 2048
N = 2048
K = 8192
TILE_M = M // N_RING
N_FWD = 4
N_BWD = 3
N_CHUNK = 4
NC = N // N_CHUNK
W_SUB = 256
SUB_PER_CHUNK = NC // W_SUB
N_SUB = N // W_SUB


def _ring_pos(x, z):
    return jnp.where(x == 0, z, (N_RING - 1) - z)


def _coords_of(rp):
    x = (rp >= 4).astype(rp.dtype)
    z = jnp.where(rp < 4, rp, (N_RING - 1) - rp)
    return x, z


def kernel(dy, W):
    my_x = lax.axis_index("x")
    my_z = lax.axis_index("z")
    pos = _ring_pos(my_x, my_z)

    dy_tile = lax.dynamic_slice_in_dim(dy, pos * TILE_M, TILE_M, axis=0)
    dy_bf = dy_tile.astype(jnp.bfloat16)

    def body(dy_ref, w_hbm, out_ref,
             w_buf, w_bf, partial,
             y_send, y_recv, red_bf, f_recv, b_recv,
             w_sems,
             y_send_sems, y_recv_sems,
             f_send_sems, f_recv_sems, b_send_sems, b_recv_sems):
        x = lax.axis_index("x")
        y = lax.axis_index("y")
        z = lax.axis_index("z")
        rp = _ring_pos(x, z)
        rx, rz = _coords_of((rp + 1) % N_RING)
        lx, lz = _coords_of((rp - 1) % N_RING)
        right = (rx, y, rz)
        left = (lx, y, lz)
        partner = (x, 1 - y, z)

        w_dmas = [None] * N_SUB
        w_dmas[0] = pltpu.make_async_copy(
            w_hbm.at[pl.ds(0, W_SUB)], w_buf.at[0], w_sems.at[0])
        w_dmas[0].start()

        barrier_sem = pltpu.get_barrier_semaphore()
        for nbr in (left, right, partner):
            pl.semaphore_signal(
                barrier_sem, inc=1,
                device_id=nbr, device_id_type=pl.DeviceIdType.MESH,
            )
        pl.semaphore_wait(barrier_sem, 3)

        def mk(src, dst, ssem, rsem, dev):
            return pltpu.make_async_remote_copy(
                src_ref=src, dst_ref=dst, send_sem=ssem, recv_sem=rsem,
                device_id=dev, device_id_type=pl.DeviceIdType.MESH,
            )

        y_rdmas = [None] * N_CHUNK
        f_rdmas = [[None] * N_CHUNK for _ in range(N_FWD)]
        b_rdmas = [[None] * N_CHUNK for _ in range(N_BWD)]

        def start_ring(c):
            y_rdmas[c].wait_recv()
            red_f32 = (partial[:, pl.ds(c * NC, NC)]
                       + y_recv[c].astype(jnp.float32))
            out_ref[pl.ds(rp * TILE_M, TILE_M), pl.ds(c * NC, NC)] = red_f32
            red_bf[c] = red_f32.astype(jnp.bfloat16)
            fr = mk(red_bf.at[c], f_recv.at[0, c],
                    f_send_sems.at[0, c], f_recv_sems.at[0, c], right)
            fr.start()
            f_rdmas[0][c] = fr
            br = mk(red_bf.at[c], b_recv.at[0, c],
                    b_send_sems.at[0, c], b_recv_sems.at[0, c], left)
            br.start()
            b_rdmas[0][c] = br

        for s in range(N_SUB):
            w_dmas[s].wait()
            if s + 1 < N_SUB:
                w_dmas[s + 1] = pltpu.make_async_copy(
                    w_hbm.at[pl.ds((s + 1) * W_SUB, W_SUB)],
                    w_buf.at[(s + 1) % 2], w_sems.at[(s + 1) % 2])
                w_dmas[s + 1].start()
            w_bf[...] = w_buf[s % 2].astype(jnp.bfloat16)
            partial[:, pl.ds(s * W_SUB, W_SUB)] = lax.dot_general(
                dy_ref[...], w_bf[...],
                dimension_numbers=(((1,), (1,)), ((), ())),
                preferred_element_type=jnp.float32,
            )
            if (s + 1) % SUB_PER_CHUNK == 0:
                c = (s + 1) // SUB_PER_CHUNK - 1
                y_send[c] = partial[:, pl.ds(c * NC, NC)].astype(jnp.bfloat16)
                r = mk(y_send.at[c], y_recv.at[c],
                       y_send_sems.at[c], y_recv_sems.at[c], partner)
                r.start()
                y_rdmas[c] = r
                if c >= 1:
                    start_ring(c - 1)
        start_ring(N_CHUNK - 1)

        for h in range(N_FWD):
            f_origin = (rp - 1 - h) % N_RING
            b_origin = (rp + 1 + h) % N_RING
            for c in range(N_CHUNK):
                f_rdmas[h][c].wait_recv()
                if h + 1 < N_FWD:
                    nxt = mk(f_recv.at[h, c], f_recv.at[h + 1, c],
                             f_send_sems.at[h + 1, c],
                             f_recv_sems.at[h + 1, c], right)
                    nxt.start()
                    f_rdmas[h + 1][c] = nxt
                out_ref[pl.ds(f_origin * TILE_M, TILE_M),
                        pl.ds(c * NC, NC)] = f_recv[h, c].astype(jnp.float32)

                if h < N_BWD:
                    b_rdmas[h][c].wait_recv()
                    if h + 1 < N_BWD:
                        nxt = mk(b_recv.at[h, c], b_recv.at[h + 1, c],
                                 b_send_sems.at[h + 1, c],
                                 b_recv_sems.at[h + 1, c], left)
                        nxt.start()
                        b_rdmas[h + 1][c] = nxt
                    out_ref[pl.ds(b_origin * TILE_M, TILE_M),
                            pl.ds(c * NC, NC)] = (
                        b_recv[h, c].astype(jnp.float32))

        for r in y_rdmas:
            r.wait_send()
        for row in f_rdmas + b_rdmas:
            for r in row:
                r.wait_send()

        @functools.partial(pl.run_scoped,
                           exit_sem=pltpu.SemaphoreType.REGULAR)
        def _(exit_sem):
            for nbr in (left, right, partner):
                pl.semaphore_signal(
                    exit_sem, inc=1,
                    device_id=nbr, device_id_type=pl.DeviceIdType.MESH,
                )
            pl.semaphore_wait(exit_sem, 3)

    return pl.pallas_call(
        body,
        out_shape=jax.ShapeDtypeStruct((M, N), jnp.float32),
        in_specs=[
            pl.BlockSpec(memory_space=pltpu.VMEM),
            pl.BlockSpec(memory_space=pltpu.MemorySpace.HBM),
        ],
        out_specs=pl.BlockSpec(memory_space=pltpu.VMEM),
        scratch_shapes=[
            pltpu.VMEM((2, W_SUB, K), jnp.float32),
            pltpu.VMEM((W_SUB, K), jnp.bfloat16),
            pltpu.VMEM((TILE_M, N), jnp.float32),
            pltpu.VMEM((N_CHUNK, TILE_M, NC), jnp.bfloat16),
            pltpu.VMEM((N_CHUNK, TILE_M, NC), jnp.bfloat16),
            pltpu.VMEM((N_CHUNK, TILE_M, NC), jnp.bfloat16),
            pltpu.VMEM((N_FWD, N_CHUNK, TILE_M, NC), jnp.bfloat16),
            pltpu.VMEM((N_BWD, N_CHUNK, TILE_M, NC), jnp.bfloat16),
            pltpu.SemaphoreType.DMA((2,)),
            pltpu.SemaphoreType.DMA((N_CHUNK,)),
            pltpu.SemaphoreType.DMA((N_CHUNK,)),
            pltpu.SemaphoreType.DMA((N_FWD, N_CHUNK)),
            pltpu.SemaphoreType.DMA((N_FWD, N_CHUNK)),
            pltpu.SemaphoreType.DMA((N_BWD, N_CHUNK)),
            pltpu.SemaphoreType.DMA((N_BWD, N_CHUNK)),
        ],
        compiler_params=pltpu.CompilerParams(collective_id=0),
    )(dy_bf, W)


# device time: 84282 ns/iter; 1.0731x vs baseline; 1.0317x over previous
import functools

import jax
import jax.numpy as jnp
from jax import lax
from jax.experimental import pallas as pl
from jax.experimental.pallas import tpu as pltpu

N_RING = 8
M = 2048
N = 2048
K = 8192
TILE_M = M // N_RING
N_FWD = 4
N_BWD = 3
N_CHUNK = 4
NC = N // N_CHUNK
W_SUB = 256
SUB_PER_CHUNK = NC // W_SUB
N_SUB = N // W_SUB


def _ring_pos(x, z):
    return jnp.where(x == 0, z, (N_RING - 1) - z)


def _coords_of(rp):
    x = (rp >= 4).astype(rp.dtype)
    z = jnp.where(rp < 4, rp, (N_RING - 1) - rp)
    return x, z


def kernel(dy, W):
    my_x = lax.axis_index("x")
    my_z = lax.axis_index("z")
    pos = _ring_pos(my_x, my_z)

    dy_tile = lax.dynamic_slice_in_dim(dy, pos * TILE_M, TILE_M, axis=0)
    dy_bf = dy_tile.astype(jnp.bfloat16)

    def body(dy_ref, w_hbm, out_ref,
             w_buf, w_bf, partial,
             y_send, y_recv, red_bf, f_recv, b_recv,
             w_sems,
             y_send_sems, y_recv_sems,
             f_send_sems, f_recv_sems, b_send_sems, b_recv_sems):
        x = lax.axis_index("x")
        y = lax.axis_index("y")
        z = lax.axis_index("z")
        rp = _ring_pos(x, z)
        rx, rz = _coords_of((rp + 1) % N_RING)
        lx, lz = _coords_of((rp - 1) % N_RING)
        right = (rx, y, rz)
        left = (lx, y, lz)
        partner = (x, 1 - y, z)

        w_dmas = [None] * N_SUB
        w_dmas[0] = pltpu.make_async_copy(
            w_hbm.at[pl.ds(0, W_SUB)], w_buf.at[0], w_sems.at[0])
        w_dmas[0].start()

        barrier_sem = pltpu.get_barrier_semaphore()
        for nbr in (left, right, partner):
            pl.semaphore_signal(
                barrier_sem, inc=1,
                device_id=nbr, device_id_type=pl.DeviceIdType.MESH,
            )
        pl.semaphore_wait(barrier_sem, 3)

        def mk(src, dst, ssem, rsem, dev):
            return pltpu.make_async_remote_copy(
                src_ref=src, dst_ref=dst, send_sem=ssem, recv_sem=rsem,
                device_id=dev, device_id_type=pl.DeviceIdType.MESH,
            )

        y_rdmas = [None] * N_CHUNK
        f_rdmas = [[None] * N_CHUNK for _ in range(N_FWD)]
        b_rdmas = [[None] * N_CHUNK for _ in range(N_BWD)]

        def start_ring(c):
            y_rdmas[c].wait_recv()
            red_f32 = (partial[:, pl.ds(c * NC, NC)]
                       + y_recv[c].astype(jnp.float32))
            out_ref[pl.ds(rp * TILE_M, TILE_M), pl.ds(c * NC, NC)] = red_f32
            red_bf[c] = red_f32.astype(jnp.bfloat16)
            fr = mk(red_bf.at[c], f_recv.at[0, c],
                    f_send_sems.at[0, c], f_recv_sems.at[0, c], right)
            fr.start()
            f_rdmas[0][c] = fr
            br = mk(red_bf.at[c], b_recv.at[0, c],
                    b_send_sems.at[0, c], b_recv_sems.at[0, c], left)
            br.start()
            b_rdmas[0][c] = br

        def ring_step(h, c):
            f_rdmas[h][c].wait_recv()
            if h + 1 < N_FWD:
                nxt = mk(f_recv.at[h, c], f_recv.at[h + 1, c],
                         f_send_sems.at[h + 1, c],
                         f_recv_sems.at[h + 1, c], right)
                nxt.start()
                f_rdmas[h + 1][c] = nxt
            f_origin = (rp - 1 - h) % N_RING
            out_ref[pl.ds(f_origin * TILE_M, TILE_M),
                    pl.ds(c * NC, NC)] = f_recv[h, c].astype(jnp.float32)

            if h < N_BWD:
                b_rdmas[h][c].wait_recv()
                if h + 1 < N_BWD:
                    nxt = mk(b_recv.at[h, c], b_recv.at[h + 1, c],
                             b_send_sems.at[h + 1, c],
                             b_recv_sems.at[h + 1, c], left)
                    nxt.start()
                    b_rdmas[h + 1][c] = nxt
                b_origin = (rp + 1 + h) % N_RING
                out_ref[pl.ds(b_origin * TILE_M, TILE_M),
                        pl.ds(c * NC, NC)] = b_recv[h, c].astype(jnp.float32)

        for s in range(N_SUB):
            w_dmas[s].wait()
            if s + 1 < N_SUB:
                w_dmas[s + 1] = pltpu.make_async_copy(
                    w_hbm.at[pl.ds((s + 1) * W_SUB, W_SUB)],
                    w_buf.at[(s + 1) % 2], w_sems.at[(s + 1) % 2])
                w_dmas[s + 1].start()
            w_bf[...] = w_buf[s % 2].astype(jnp.bfloat16)
            partial[:, pl.ds(s * W_SUB, W_SUB)] = lax.dot_general(
                dy_ref[...], w_bf[...],
                dimension_numbers=(((1,), (1,)), ((), ())),
                preferred_element_type=jnp.float32,
            )
            if (s + 1) % SUB_PER_CHUNK == 0:
                c = (s + 1) // SUB_PER_CHUNK - 1
                y_send[c] = partial[:, pl.ds(c * NC, NC)].astype(jnp.bfloat16)
                r = mk(y_send.at[c], y_recv.at[c],
                       y_send_sems.at[c], y_recv_sems.at[c], partner)
                r.start()
                y_rdmas[c] = r
                if c >= 1:
                    start_ring(c - 1)
                if c == 2:
                    ring_step(0, 0)
                elif c == 3:
                    ring_step(0, 1)
                    ring_step(1, 0)
        start_ring(N_CHUNK - 1)

        done = {(0, 0), (0, 1), (1, 0)}
        for k in range(N_FWD + N_CHUNK - 1):
            for h in range(min(k, N_FWD - 1), -1, -1):
                c = k - h
                if c < 0 or c >= N_CHUNK or (h, c) in done:
                    continue
                ring_step(h, c)

        for r in y_rdmas:
            r.wait_send()
        for row in f_rdmas + b_rdmas:
            for r in row:
                r.wait_send()

        @functools.partial(pl.run_scoped,
                           exit_sem=pltpu.SemaphoreType.REGULAR)
        def _(exit_sem):
            for nbr in (left, right, partner):
                pl.semaphore_signal(
                    exit_sem, inc=1,
                    device_id=nbr, device_id_type=pl.DeviceIdType.MESH,
                )
            pl.semaphore_wait(exit_sem, 3)

    return pl.pallas_call(
        body,
        out_shape=jax.ShapeDtypeStruct((M, N), jnp.float32),
        in_specs=[
            pl.BlockSpec(memory_space=pltpu.VMEM),
            pl.BlockSpec(memory_space=pltpu.MemorySpace.HBM),
        ],
        out_specs=pl.BlockSpec(memory_space=pltpu.VMEM),
        scratch_shapes=[
            pltpu.VMEM((2, W_SUB, K), jnp.float32),
            pltpu.VMEM((W_SUB, K), jnp.bfloat16),
            pltpu.VMEM((TILE_M, N), jnp.float32),
            pltpu.VMEM((N_CHUNK, TILE_M, NC), jnp.bfloat16),
            pltpu.VMEM((N_CHUNK, TILE_M, NC), jnp.bfloat16),
            pltpu.VMEM((N_CHUNK, TILE_M, NC), jnp.bfloat16),
            pltpu.VMEM((N_FWD, N_CHUNK, TILE_M, NC), jnp.bfloat16),
            pltpu.VMEM((N_BWD, N_CHUNK, TILE_M, NC), jnp.bfloat16),
            pltpu.SemaphoreType.DMA((2,)),
            pltpu.SemaphoreType.DMA((N_CHUNK,)),
            pltpu.SemaphoreType.DMA((N_CHUNK,)),
            pltpu.SemaphoreType.DMA((N_FWD, N_CHUNK)),
            pltpu.SemaphoreType.DMA((N_FWD, N_CHUNK)),
            pltpu.SemaphoreType.DMA((N_BWD, N_CHUNK)),
            pltpu.SemaphoreType.DMA((N_BWD, N_CHUNK)),
        ],
        compiler_params=pltpu.CompilerParams(collective_id=0),
    )(dy_bf, W)


# device time: 44104 ns/iter; 2.0507x vs baseline; 1.9110x over previous
import functools
import os

import jax
import jax.numpy as jnp
from jax import lax
from jax.experimental import pallas as pl
from jax.experimental.pallas import tpu as pltpu

KMODE = os.environ.get("KMODE", "full")

N_RING = 8
M = 2048
N = 2048
K = 8192
TILE_M = M // N_RING
N_FWD = 4
N_BWD = 3
N_CHUNK = 4
NC = N // N_CHUNK
W_SUB = 256
SUB_PER_CHUNK = NC // W_SUB
N_SUB = N // W_SUB


def _ring_pos(x, z):
    return jnp.where(x == 0, z, (N_RING - 1) - z)


def _coords_of(rp):
    x = (rp >= 4).astype(rp.dtype)
    z = jnp.where(rp < 4, rp, (N_RING - 1) - rp)
    return x, z


def kernel(dy, W):
    my_x = lax.axis_index("x")
    my_z = lax.axis_index("z")
    pos = _ring_pos(my_x, my_z)

    dy_tile = lax.dynamic_slice_in_dim(dy, pos * TILE_M, TILE_M, axis=0)
    dy_bf = dy_tile.astype(jnp.bfloat16)

    def body(dy_ref, w_hbm, out_ref,
             w_buf, w_bf, partial,
             y_send, y_recv, red_bf, f_recv, b_recv,
             w_sems,
             y_send_sems, y_recv_sems,
             f_send_sems, f_recv_sems, b_send_sems, b_recv_sems):
        x = lax.axis_index("x")
        y = lax.axis_index("y")
        z = lax.axis_index("z")
        rp = _ring_pos(x, z)
        rx, rz = _coords_of((rp + 1) % N_RING)
        lx, lz = _coords_of((rp - 1) % N_RING)
        right = (rx, y, rz)
        left = (lx, y, lz)
        partner = (x, 1 - y, z)

        w_dmas = [None] * N_SUB
        if KMODE != "comm":
            w_dmas[0] = pltpu.make_async_copy(
                w_hbm.at[pl.ds(0, W_SUB)], w_buf.at[0], w_sems.at[0])
            w_dmas[0].start()

        if KMODE != "gemm":
            barrier_sem = pltpu.get_barrier_semaphore()
            for nbr in (left, right, partner):
                pl.semaphore_signal(
                    barrier_sem, inc=1,
                    device_id=nbr, device_id_type=pl.DeviceIdType.MESH,
                )
            pl.semaphore_wait(barrier_sem, 3)

        def mk(src, dst, ssem, rsem, dev):
            return pltpu.make_async_remote_copy(
                src_ref=src, dst_ref=dst, send_sem=ssem, recv_sem=rsem,
                device_id=dev, device_id_type=pl.DeviceIdType.MESH,
            )

        y_rdmas = [None] * N_CHUNK
        f_rdmas = [[None] * N_CHUNK for _ in range(N_FWD)]
        b_rdmas = [[None] * N_CHUNK for _ in range(N_BWD)]

        def start_ring(c):
            y_rdmas[c].wait_recv()
            red_f32 = (partial[:, pl.ds(c * NC, NC)]
                       + y_recv[c].astype(jnp.float32))
            out_ref[pl.ds(rp * TILE_M, TILE_M), pl.ds(c * NC, NC)] = red_f32
            red_bf[c] = red_f32.astype(jnp.bfloat16)
            fr = mk(red_bf.at[c], f_recv.at[0, c],
                    f_send_sems.at[0, c], f_recv_sems.at[0, c], right)
            fr.start()
            f_rdmas[0][c] = fr
            br = mk(red_bf.at[c], b_recv.at[0, c],
                    b_send_sems.at[0, c], b_recv_sems.at[0, c], left)
            br.start()
            b_rdmas[0][c] = br

        def ring_step(h, c):
            f_rdmas[h][c].wait_recv()
            if h + 1 < N_FWD:
                nxt = mk(f_recv.at[h, c], f_recv.at[h + 1, c],
                         f_send_sems.at[h + 1, c],
                         f_recv_sems.at[h + 1, c], right)
                nxt.start()
                f_rdmas[h + 1][c] = nxt
            f_origin = (rp - 1 - h) % N_RING
            out_ref[pl.ds(f_origin * TILE_M, TILE_M),
                    pl.ds(c * NC, NC)] = f_recv[h, c].astype(jnp.float32)

            if h < N_BWD:
                b_rdmas[h][c].wait_recv()
                if h + 1 < N_BWD:
                    nxt = mk(b_recv.at[h, c], b_recv.at[h + 1, c],
                             b_send_sems.at[h + 1, c],
                             b_recv_sems.at[h + 1, c], left)
                    nxt.start()
                    b_rdmas[h + 1][c] = nxt
                b_origin = (rp + 1 + h) % N_RING
                out_ref[pl.ds(b_origin * TILE_M, TILE_M),
                        pl.ds(c * NC, NC)] = b_recv[h, c].astype(jnp.float32)

        if KMODE == "comm":
            partial[...] = jnp.zeros_like(partial)
        else:
            for s in range(N_SUB):
                w_dmas[s].wait()
                if s + 1 < N_SUB:
                    w_dmas[s + 1] = pltpu.make_async_copy(
                        w_hbm.at[pl.ds((s + 1) * W_SUB, W_SUB)],
                        w_buf.at[(s + 1) % 2], w_sems.at[(s + 1) % 2])
                    w_dmas[s + 1].start()
                w_bf[...] = w_buf[s % 2].astype(jnp.bfloat16)
                partial[:, pl.ds(s * W_SUB, W_SUB)] = lax.dot_general(
                    dy_ref[...], w_bf[...],
                    dimension_numbers=(((1,), (1,)), ((), ())),
                    preferred_element_type=jnp.float32,
                )
                if KMODE == "full" and (s + 1) % SUB_PER_CHUNK == 0:
                    c = (s + 1) // SUB_PER_CHUNK - 1
                    y_send[c] = (
                        partial[:, pl.ds(c * NC, NC)].astype(jnp.bfloat16))
                    r = mk(y_send.at[c], y_recv.at[c],
                           y_send_sems.at[c], y_recv_sems.at[c], partner)
                    r.start()
                    y_rdmas[c] = r
                    if c >= 1:
                        start_ring(c - 1)
                    if c == 2:
                        ring_step(0, 0)
                    elif c == 3:
                        ring_step(0, 1)
                        ring_step(1, 0)

        if KMODE == "gemm":
            out_ref[pl.ds(rp * TILE_M, TILE_M), :] = partial[...]
            return

        if KMODE == "comm":
            done = set()
            for c in range(N_CHUNK):
                y_send[c] = partial[:, pl.ds(c * NC, NC)].astype(jnp.bfloat16)
                r = mk(y_send.at[c], y_recv.at[c],
                       y_send_sems.at[c], y_recv_sems.at[c], partner)
                r.start()
                y_rdmas[c] = r
                if c >= 1:
                    start_ring(c - 1)
        else:
            done = {(0, 0), (0, 1), (1, 0)}
        start_ring(N_CHUNK - 1)

        for k in range(N_FWD + N_CHUNK - 1):
            for h in range(min(k, N_FWD - 1), -1, -1):
                c = k - h
                if c < 0 or c >= N_CHUNK or (h, c) in done:
                    continue
                ring_step(h, c)

        for r in y_rdmas:
            r.wait_send()
        for row in f_rdmas + b_rdmas:
            for r in row:
                r.wait_send()

        @functools.partial(pl.run_scoped,
                           exit_sem=pltpu.SemaphoreType.REGULAR)
        def _(exit_sem):
            for nbr in (left, right, partner):
                pl.semaphore_signal(
                    exit_sem, inc=1,
                    device_id=nbr, device_id_type=pl.DeviceIdType.MESH,
                )
            pl.semaphore_wait(exit_sem, 3)

    return pl.pallas_call(
        body,
        out_shape=jax.ShapeDtypeStruct((M, N), jnp.float32),
        in_specs=[
            pl.BlockSpec(memory_space=pltpu.VMEM),
            pl.BlockSpec(memory_space=pltpu.MemorySpace.HBM),
        ],
        out_specs=pl.BlockSpec(memory_space=pltpu.VMEM),
        scratch_shapes=[
            pltpu.VMEM((2, W_SUB, K), jnp.float32),
            pltpu.VMEM((W_SUB, K), jnp.bfloat16),
            pltpu.VMEM((TILE_M, N), jnp.float32),
            pltpu.VMEM((N_CHUNK, TILE_M, NC), jnp.bfloat16),
            pltpu.VMEM((N_CHUNK, TILE_M, NC), jnp.bfloat16),
            pltpu.VMEM((N_CHUNK, TILE_M, NC), jnp.bfloat16),
            pltpu.VMEM((N_FWD, N_CHUNK, TILE_M, NC), jnp.bfloat16),
            pltpu.VMEM((N_BWD, N_CHUNK, TILE_M, NC), jnp.bfloat16),
            pltpu.SemaphoreType.DMA((2,)),
            pltpu.SemaphoreType.DMA((N_CHUNK,)),
            pltpu.SemaphoreType.DMA((N_CHUNK,)),
            pltpu.SemaphoreType.DMA((N_FWD, N_CHUNK)),
            pltpu.SemaphoreType.DMA((N_FWD, N_CHUNK)),
            pltpu.SemaphoreType.DMA((N_BWD, N_CHUNK)),
            pltpu.SemaphoreType.DMA((N_BWD, N_CHUNK)),
        ],
        compiler_params=(
            pltpu.CompilerParams(vmem_limit_bytes=120 * 2**20)
            if KMODE == "gemm"
            else pltpu.CompilerParams(collective_id=0)),
    )(dy_bf, W)
